# Initial kernel scaffold; baseline (speedup 1.0000x reference)
import jax
import jax.numpy as jnp
from jax.experimental import pallas as pl


def kernel(x, edge_index, edge_weight, W1, b1, W2, b2, Wout, bout):
    raise NotImplementedError("write your pallas kernel here")


# speedup vs baseline: 8.9516x; 8.9516x over previous
import functools

import jax
import jax.numpy as jnp
from jax import lax
from jax.experimental import pallas as pl
from jax.experimental.pallas import tpu as pltpu
from jax.experimental.pallas import tpu_sc as plsc

N = 10000
E = 320000
D = 128
C = 64
NC, NS, LANES = 2, 16, 16
NW = NC * NS
EPW = E // NW
K = 80
NCHUNK = EPW // K
NPAD = 10240
NPT = NPAD // NS
RPT = N // NS
MMB = 400

_SC_PARAMS = pltpu.CompilerParams(needs_layout_passes=False)


def _mesh():
    return plsc.VectorSubcoreMesh(core_axis_name="c", subcore_axis_name="s")


@functools.partial(
    pl.kernel,
    out_type=jax.ShapeDtypeStruct((NC, NPAD), jnp.float32),
    mesh=_mesh(),
    compiler_params=_SC_PARAMS,
    scratch_types=[
        pltpu.VMEM((NPAD,), jnp.float32),
        pltpu.VMEM((K,), jnp.int32),
        pltpu.VMEM((K,), jnp.float32),
        pltpu.VMEM((NS, NPT), jnp.float32),
        pltpu.VMEM_SHARED((NS, NPAD), jnp.float32),
    ],
)
def _deg_kernel(dst_hbm, w_hbm, degp, acc, dbuf, wbuf, rbuf, shared):
    cid = lax.axis_index("c")
    sid = lax.axis_index("s")
    wid = cid * NS + sid

    def zero_body(i, c):
        acc[pl.ds(i * LANES, LANES)] = jnp.zeros((LANES,), jnp.float32)
        return c

    lax.fori_loop(0, NPAD // LANES, zero_body, 0)

    def chunk_body(c, carry):
        base = pl.multiple_of(wid * EPW + c * K, 8)
        pltpu.sync_copy(dst_hbm.at[pl.ds(base, K)], dbuf)
        pltpu.sync_copy(w_hbm.at[pl.ds(base, K)], wbuf)
        for i in range(K // LANES):
            idx = dbuf[pl.ds(i * LANES, LANES)]
            val = wbuf[pl.ds(i * LANES, LANES)]
            plsc.addupdate_scatter(acc, [idx], val)
        return carry

    lax.fori_loop(0, NCHUNK, chunk_body, 0)

    pltpu.sync_copy(acc, shared.at[sid])
    plsc.subcore_barrier()
    col0 = pl.multiple_of(sid * NPT, 8)
    for r in range(NS):
        pltpu.sync_copy(shared.at[r, pl.ds(col0, NPT)], rbuf.at[r])

    def red_body(i, carry):
        s = rbuf[0, pl.ds(i * LANES, LANES)]
        for r in range(1, NS):
            s = s + rbuf[r, pl.ds(i * LANES, LANES)]
        acc[pl.ds(i * LANES, LANES)] = s
        return carry

    lax.fori_loop(0, NPT // LANES, red_body, 0)
    pltpu.sync_copy(acc.at[pl.ds(0, NPT)], degp.at[cid, pl.ds(col0, NPT)])


def _prop_common(cid, sid, wid, h_hbm, part, get_norm_chunk,
                 sbuf, dbuf, nbuf, rows, zbuf, acc, sem):
    def zfill(i, c):
        for j in range(D // LANES):
            zbuf[i, pl.ds(j * LANES, LANES)] = jnp.zeros((LANES,), jnp.float32)
        return c

    lax.fori_loop(0, 128, zfill, 0)
    row0 = pl.multiple_of(sid * NPT, 8)
    for k in range(NPT // 128):
        pltpu.sync_copy(zbuf, acc.at[pl.ds(row0 + k * 128, 128)])
    plsc.subcore_barrier()

    def chunk_body(c, carry):
        base = pl.multiple_of(wid * EPW + c * K, 8)
        pltpu.sync_copy(_src_hbm_ref[0].at[pl.ds(base, K)], sbuf)
        pltpu.sync_copy(_dst_hbm_ref[0].at[pl.ds(base, K)], dbuf)
        get_norm_chunk(base)
        pltpu.async_copy(h_hbm.at[sbuf], rows, sem).wait()

        def scale_body(e, cc):
            s = plsc.load_gather(nbuf, [jnp.zeros((LANES,), jnp.int32) + e])
            for j in range(D // LANES):
                rows[e, pl.ds(j * LANES, LANES)] = (
                    rows[e, pl.ds(j * LANES, LANES)] * s)
            return cc

        lax.fori_loop(0, K, scale_body, 0)
        pltpu.sync_copy(rows, acc.at[dbuf], add=True)
        return carry

    lax.fori_loop(0, NCHUNK, chunk_body, 0)
    plsc.subcore_barrier()
    pltpu.sync_copy(acc.at[pl.ds(row0, NPT)],
                    part.at[cid, pl.ds(row0, NPT)])


_src_hbm_ref = [None]
_dst_hbm_ref = [None]


@functools.partial(
    pl.kernel,
    out_type=(jax.ShapeDtypeStruct((NC, NPAD, D), jnp.float32),
              jax.ShapeDtypeStruct((E,), jnp.float32)),
    mesh=_mesh(),
    compiler_params=_SC_PARAMS,
    scratch_types=[
        pltpu.VMEM((K,), jnp.int32),
        pltpu.VMEM((K,), jnp.int32),
        pltpu.VMEM((K,), jnp.float32),
        pltpu.VMEM((K,), jnp.float32),
        pltpu.VMEM((NPAD,), jnp.float32),
        pltpu.VMEM((K, D), jnp.float32),
        pltpu.VMEM((128, D), jnp.float32),
        pltpu.VMEM_SHARED((NPAD, D), jnp.float32),
        pltpu.SemaphoreType.DMA,
    ],
)
def _prop1_kernel(h_hbm, src_hbm, dst_hbm, w_hbm, dinv_hbm,
                  part, norm_out,
                  sbuf, dbuf, nbuf, wbuf, dv, rows, zbuf, acc, sem):
    cid = lax.axis_index("c")
    sid = lax.axis_index("s")
    wid = cid * NS + sid
    pltpu.sync_copy(dinv_hbm, dv)
    _src_hbm_ref[0] = src_hbm
    _dst_hbm_ref[0] = dst_hbm

    def get_norm_chunk(base):
        pltpu.sync_copy(w_hbm.at[pl.ds(base, K)], wbuf)
        for i in range(K // LANES):
            s16 = sbuf[pl.ds(i * LANES, LANES)]
            d16 = dbuf[pl.ds(i * LANES, LANES)]
            nv = (plsc.load_gather(dv, [s16]) * wbuf[pl.ds(i * LANES, LANES)]
                  * plsc.load_gather(dv, [d16]))
            nbuf[pl.ds(i * LANES, LANES)] = nv
        pltpu.sync_copy(nbuf, norm_out.at[pl.ds(base, K)])

    _prop_common(cid, sid, wid, h_hbm, part, get_norm_chunk,
                 sbuf, dbuf, nbuf, rows, zbuf, acc, sem)


@functools.partial(
    pl.kernel,
    out_type=jax.ShapeDtypeStruct((NC, NPAD, D), jnp.float32),
    mesh=_mesh(),
    compiler_params=_SC_PARAMS,
    scratch_types=[
        pltpu.VMEM((K,), jnp.int32),
        pltpu.VMEM((K,), jnp.int32),
        pltpu.VMEM((K,), jnp.float32),
        pltpu.VMEM((K, D), jnp.float32),
        pltpu.VMEM((128, D), jnp.float32),
        pltpu.VMEM_SHARED((NPAD, D), jnp.float32),
        pltpu.SemaphoreType.DMA,
    ],
)
def _prop2_kernel(h_hbm, src_hbm, dst_hbm, norm_hbm,
                  part,
                  sbuf, dbuf, nbuf, rows, zbuf, acc, sem):
    cid = lax.axis_index("c")
    sid = lax.axis_index("s")
    wid = cid * NS + sid
    _src_hbm_ref[0] = src_hbm
    _dst_hbm_ref[0] = dst_hbm

    def get_norm_chunk(base):
        pltpu.sync_copy(norm_hbm.at[pl.ds(base, K)], nbuf)

    _prop_common(cid, sid, wid, h_hbm, part, get_norm_chunk,
                 sbuf, dbuf, nbuf, rows, zbuf, acc, sem)


def _prep_body(degp_ref, dinv_ref, dinv2_ref):
    deg = degp_ref[0] + degp_ref[1] + 1.0
    dinv_ref[...] = lax.rsqrt(deg)
    dinv2_ref[...] = 1.0 / deg


_prep = pl.pallas_call(
    _prep_body,
    out_shape=(jax.ShapeDtypeStruct((NPAD // 128, 128), jnp.float32),
               jax.ShapeDtypeStruct((NPAD // 128, 128), jnp.float32)),
)


def _mm_body(a_ref, w_ref, o_ref):
    o_ref[...] = jnp.dot(a_ref[...], w_ref[...],
                         preferred_element_type=jnp.float32,
                         precision=lax.Precision.HIGHEST)


_mm = pl.pallas_call(
    _mm_body,
    grid=(N // MMB,),
    in_specs=[pl.BlockSpec((MMB, D), lambda i: (i, 0)),
              pl.BlockSpec((D, D), lambda i: (0, 0))],
    out_specs=pl.BlockSpec((MMB, D), lambda i: (i, 0)),
    out_shape=jax.ShapeDtypeStruct((N, D), jnp.float32),
)


def _fuse_mm_body(p0_ref, p1_ref, h_ref, dv2_ref, b_ref, w_ref, o_ref):
    a = (p0_ref[0] + p1_ref[0] + dv2_ref[...] * h_ref[...] + b_ref[...])
    a = jnp.maximum(a, 0.0)
    o_ref[...] = jnp.dot(a, w_ref[...],
                         preferred_element_type=jnp.float32,
                         precision=lax.Precision.HIGHEST)


_fuse_mm = pl.pallas_call(
    _fuse_mm_body,
    grid=(N // MMB,),
    in_specs=[pl.BlockSpec((1, MMB, D), lambda i: (0, i, 0)),
              pl.BlockSpec((1, MMB, D), lambda i: (1, i, 0)),
              pl.BlockSpec((MMB, D), lambda i: (i, 0)),
              pl.BlockSpec((MMB, 1), lambda i: (i, 0)),
              pl.BlockSpec((1, D), lambda i: (0, 0)),
              pl.BlockSpec((D, D), lambda i: (0, 0))],
    out_specs=pl.BlockSpec((MMB, D), lambda i: (i, 0)),
    out_shape=jax.ShapeDtypeStruct((N, D), jnp.float32),
)


def _final_body(p0_ref, p1_ref, h_ref, dv2_ref, b_ref, w_ref, bo_ref, o_ref):
    a = (p0_ref[0] + p1_ref[0] + dv2_ref[...] * h_ref[...] + b_ref[...])
    a = jnp.maximum(a, 0.0)
    logits = jnp.dot(a, w_ref[...],
                     preferred_element_type=jnp.float32,
                     precision=lax.Precision.HIGHEST) + bo_ref[...]
    col = lax.broadcasted_iota(jnp.int32, (MMB, 128), 1)
    lm = jnp.where(col < C, logits, jnp.float32(-1e30))
    m = jnp.max(lm, axis=1, keepdims=True)
    ex = jnp.where(col < C, jnp.exp(lm - m), 0.0)
    o_ref[...] = ex / jnp.sum(ex, axis=1, keepdims=True)


_final = pl.pallas_call(
    _final_body,
    grid=(N // MMB,),
    in_specs=[pl.BlockSpec((1, MMB, D), lambda i: (0, i, 0)),
              pl.BlockSpec((1, MMB, D), lambda i: (1, i, 0)),
              pl.BlockSpec((MMB, D), lambda i: (i, 0)),
              pl.BlockSpec((MMB, 1), lambda i: (i, 0)),
              pl.BlockSpec((1, D), lambda i: (0, 0)),
              pl.BlockSpec((D, 128), lambda i: (0, 0)),
              pl.BlockSpec((1, 128), lambda i: (0, 0))],
    out_specs=pl.BlockSpec((MMB, 128), lambda i: (i, 0)),
    out_shape=jax.ShapeDtypeStruct((N, 128), jnp.float32),
)


def kernel(x, edge_index, edge_weight, W1, b1, W2, b2, Wout, bout):
    src = edge_index[0].astype(jnp.int32)
    dst = edge_index[1].astype(jnp.int32)
    w = edge_weight.astype(jnp.float32)

    degp = _deg_kernel(dst, w)
    dinv, dinv2 = _prep(degp.reshape(NC, NPAD // 128, 128))
    dinv_flat = dinv.reshape(NPAD)
    dinv2_col = dinv2.reshape(NPAD)[:N, None]

    h1 = _mm(x, W1)
    part1, norm = _prop1_kernel(h1, src, dst, w, dinv_flat)
    h2 = _fuse_mm(part1, part1, h1, dinv2_col, b1.reshape(1, D), W2)
    part2 = _prop2_kernel(h2, src, dst, norm)
    Wout_pad = jnp.pad(Wout, ((0, 0), (0, 128 - C)))
    bout_pad = jnp.pad(bout, (0, 128 - C)).reshape(1, 128)
    outp = _final(part2, part2, h2, dinv2_col, b2.reshape(1, D),
                  Wout_pad, bout_pad)
    return outp[:, :C]


# speedup vs baseline: 9.1228x; 1.0191x over previous
import functools

import jax
import jax.numpy as jnp
from jax import lax
from jax.experimental import pallas as pl
from jax.experimental.pallas import tpu as pltpu
from jax.experimental.pallas import tpu_sc as plsc

N = 10000
E = 320000
D = 128
C = 64
NC, NS, LANES = 2, 16, 16
NW = NC * NS
K = 64
CPW = 160
R = 4
EPW = K * CPW
EPAD = EPW * NW
NCH_ALL = EPAD // K
NPAD = 10240
NPT = NPAD // NS
MMB = 400

_SC_PARAMS = pltpu.CompilerParams(needs_layout_passes=False)


def _mesh():
    return plsc.VectorSubcoreMesh(core_axis_name="c", subcore_axis_name="s")


def _splat(v):
    return jnp.zeros((LANES,), jnp.int32) + v


@functools.partial(
    pl.kernel,
    out_type=jax.ShapeDtypeStruct((NC, NPAD), jnp.float32),
    mesh=_mesh(),
    compiler_params=_SC_PARAMS,
    scratch_types=[
        pltpu.VMEM((NPAD,), jnp.float32),
        pltpu.VMEM((CPW, K), jnp.int32),
        pltpu.VMEM((CPW, K), jnp.float32),
        pltpu.VMEM((NS, NPT), jnp.float32),
        pltpu.VMEM_SHARED((NS, NPAD), jnp.float32),
    ],
)
def _deg_kernel(dpack_hbm, wpack_hbm, degp, acc, dbig, wbig, rbuf, shared):
    cid = lax.axis_index("c")
    sid = lax.axis_index("s")
    wid = cid * NS + sid

    def zero_body(i, c):
        acc[pl.ds(i * LANES, LANES)] = jnp.zeros((LANES,), jnp.float32)
        return c

    lax.fori_loop(0, NPAD // LANES, zero_body, 0)
    slab = pl.ds(pl.multiple_of(wid * CPW, 8), CPW)
    pltpu.sync_copy(dpack_hbm.at[slab], dbig)
    pltpu.sync_copy(wpack_hbm.at[slab], wbig)

    def chunk_body(c, carry):
        for i in range(K // LANES):
            idx = dbig[c, pl.ds(i * LANES, LANES)]
            val = wbig[c, pl.ds(i * LANES, LANES)]
            plsc.addupdate_scatter(acc, [idx], val)
        return carry

    lax.fori_loop(0, CPW, chunk_body, 0)

    pltpu.sync_copy(acc, shared.at[sid])
    plsc.subcore_barrier()
    col0 = pl.multiple_of(sid * NPT, 8)
    for r in range(NS):
        pltpu.sync_copy(shared.at[r, pl.ds(col0, NPT)], rbuf.at[r])

    def red_body(i, carry):
        s = rbuf[0, pl.ds(i * LANES, LANES)]
        for r in range(1, NS):
            s = s + rbuf[r, pl.ds(i * LANES, LANES)]
        acc[pl.ds(i * LANES, LANES)] = s
        return carry

    lax.fori_loop(0, NPT // LANES, red_body, 0)
    pltpu.sync_copy(acc.at[pl.ds(0, NPT)], degp.at[cid, pl.ds(col0, NPT)])


@functools.partial(
    pl.kernel,
    out_type=jax.ShapeDtypeStruct((NC, NPAD, D), jnp.float32),
    mesh=_mesh(),
    compiler_params=_SC_PARAMS,
    scratch_types=[
        [pltpu.VMEM((3, K), jnp.int32) for _ in range(R)],
        [pltpu.VMEM((K, D), jnp.float32) for _ in range(R)],
        pltpu.VMEM_SHARED((NPAD, D), jnp.float32),
        [pltpu.SemaphoreType.DMA for _ in range(R)],
        [pltpu.SemaphoreType.DMA for _ in range(R)],
        [pltpu.SemaphoreType.DMA for _ in range(R)],
    ],
)
def _prop_kernel(h_hbm, sdwpack_hbm, part, sdw, rows, acc, esem, gsem, ssem):
    cid = lax.axis_index("c")
    sid = lax.axis_index("s")
    wid = cid * NS + sid

    def zfill(i, c):
        for j in range(D // LANES):
            rows[0][i, pl.ds(j * LANES, LANES)] = jnp.zeros((LANES,),
                                                            jnp.float32)
        return c

    lax.fori_loop(0, K, zfill, 0)
    row0 = pl.multiple_of(sid * NPT, 8)

    def zcopy(k, carry):
        pltpu.sync_copy(rows[0],
                        acc.at[pl.ds(pl.multiple_of(row0 + k * K, 8), K)])
        return carry

    lax.fori_loop(0, NPT // K, zcopy, 0)
    plsc.subcore_barrier()

    gbase = wid * CPW

    def sw_start(c, b):
        pltpu.async_copy(sdwpack_hbm.at[gbase + c], sdw[b], esem[b])

    def sw_wait(c, b):
        pltpu.make_async_copy(sdwpack_hbm.at[gbase + c], sdw[b],
                              esem[b]).wait()

    def gather_start(c, b):
        pltpu.async_copy(h_hbm.at[sdw[b].at[0]], rows[b], gsem[b])

    def gather_wait(c, b):
        pltpu.make_async_copy(h_hbm.at[sdw[b].at[0]], rows[b],
                              gsem[b]).wait()

    def scatter_start(c, b):
        pltpu.async_copy(rows[b], acc.at[sdw[b].at[1]], ssem[b], add=True)

    def scatter_wait(c, b):
        pltpu.make_async_copy(rows[b], acc.at[sdw[b].at[1]], ssem[b]).wait()

    def process(c, b):
        gather_wait(c, b)
        two16 = _splat(2)

        def sbody(e, cc):
            sw = plsc.load_gather(sdw[b], [two16, _splat(e)])
            sf = plsc.bitcast(sw, jnp.float32)
            for j in range(D // LANES):
                rows[b][e, pl.ds(j * LANES, LANES)] = (
                    rows[b][e, pl.ds(j * LANES, LANES)] * sf)
            return cc

        lax.fori_loop(0, K, sbody, 0)
        scatter_start(c, b)

    for j in range(R):
        sw_start(j, j)
    sw_wait(0, 0)
    gather_start(0, 0)
    sw_wait(1, 1)
    gather_start(1, 1)

    def loop_body(i, carry):
        for u in range(R):
            c = i * R + u
            b = u
            b2 = (u + 2) % R
            b3 = (u + 3) % R

            process(c, b)

            @pl.when(c + 2 < CPW)
            def _():
                sw_wait(c + 2, b2)
                gather_start(c + 2, b2)

            @pl.when(jnp.logical_and(c + 3 > R - 1, c + 3 < CPW))
            def _():
                scatter_wait(c - 1, b3)
                sw_start(c + 3, b3)
        return carry

    lax.fori_loop(0, CPW // R, loop_body, 0)
    for b in range(R):
        scatter_wait(CPW - R + b, b)
    plsc.subcore_barrier()

    def wb_body(k, carry):
        r = pl.multiple_of(row0 + k * 64, 8)
        pltpu.sync_copy(acc.at[pl.ds(r, 64)], part.at[cid, pl.ds(r, 64)])
        return carry

    lax.fori_loop(0, NPT // 64, wb_body, 0)


def _prep_body(degp_ref, dinv_ref):
    deg = degp_ref[0] + degp_ref[1] + 1.0
    dinv_ref[...] = lax.rsqrt(deg)


_prep = pl.pallas_call(
    _prep_body,
    out_shape=jax.ShapeDtypeStruct((NPAD // 128, 128), jnp.float32),
)


def _mm_body(a_ref, w_ref, dv_ref, o_ref):
    o_ref[...] = dv_ref[...] * jnp.dot(a_ref[...], w_ref[...],
                                       preferred_element_type=jnp.float32,
                                       precision=lax.Precision.HIGHEST)


_mm = pl.pallas_call(
    _mm_body,
    grid=(N // MMB,),
    in_specs=[pl.BlockSpec((MMB, D), lambda i: (i, 0)),
              pl.BlockSpec((D, D), lambda i: (0, 0)),
              pl.BlockSpec((MMB, 1), lambda i: (i, 0))],
    out_specs=pl.BlockSpec((MMB, D), lambda i: (i, 0)),
    out_shape=jax.ShapeDtypeStruct((N, D), jnp.float32),
)


def _fuse_mm_body(p0_ref, p1_ref, h_ref, dv_ref, b_ref, w_ref, o_ref):
    a = dv_ref[...] * (p0_ref[0] + p1_ref[0] + h_ref[...]) + b_ref[...]
    a = jnp.maximum(a, 0.0)
    o_ref[...] = dv_ref[...] * jnp.dot(a, w_ref[...],
                                       preferred_element_type=jnp.float32,
                                       precision=lax.Precision.HIGHEST)


_fuse_mm = pl.pallas_call(
    _fuse_mm_body,
    grid=(N // MMB,),
    in_specs=[pl.BlockSpec((1, MMB, D), lambda i: (0, i, 0)),
              pl.BlockSpec((1, MMB, D), lambda i: (1, i, 0)),
              pl.BlockSpec((MMB, D), lambda i: (i, 0)),
              pl.BlockSpec((MMB, 1), lambda i: (i, 0)),
              pl.BlockSpec((1, D), lambda i: (0, 0)),
              pl.BlockSpec((D, D), lambda i: (0, 0))],
    out_specs=pl.BlockSpec((MMB, D), lambda i: (i, 0)),
    out_shape=jax.ShapeDtypeStruct((N, D), jnp.float32),
)


def _final_body(p0_ref, p1_ref, h_ref, dv_ref, b_ref, w_ref, bo_ref, o_ref):
    a = dv_ref[...] * (p0_ref[0] + p1_ref[0] + h_ref[...]) + b_ref[...]
    a = jnp.maximum(a, 0.0)
    logits = jnp.dot(a, w_ref[...],
                     preferred_element_type=jnp.float32,
                     precision=lax.Precision.HIGHEST) + bo_ref[...]
    col = lax.broadcasted_iota(jnp.int32, (MMB, 128), 1)
    lm = jnp.where(col < C, logits, jnp.float32(-1e30))
    m = jnp.max(lm, axis=1, keepdims=True)
    ex = jnp.where(col < C, jnp.exp(lm - m), 0.0)
    o_ref[...] = ex / jnp.sum(ex, axis=1, keepdims=True)


_final = pl.pallas_call(
    _final_body,
    grid=(N // MMB,),
    in_specs=[pl.BlockSpec((1, MMB, D), lambda i: (0, i, 0)),
              pl.BlockSpec((1, MMB, D), lambda i: (1, i, 0)),
              pl.BlockSpec((MMB, D), lambda i: (i, 0)),
              pl.BlockSpec((MMB, 1), lambda i: (i, 0)),
              pl.BlockSpec((1, D), lambda i: (0, 0)),
              pl.BlockSpec((D, 128), lambda i: (0, 0)),
              pl.BlockSpec((1, 128), lambda i: (0, 0))],
    out_specs=pl.BlockSpec((MMB, 128), lambda i: (i, 0)),
    out_shape=jax.ShapeDtypeStruct((N, 128), jnp.float32),
)


def kernel(x, edge_index, edge_weight, W1, b1, W2, b2, Wout, bout):
    src = edge_index[0].astype(jnp.int32)
    dst = edge_index[1].astype(jnp.int32)
    w = edge_weight.astype(jnp.float32)
    pad = EPAD - E
    spack = jnp.concatenate([src, jnp.zeros((pad,), jnp.int32)]).reshape(
        NCH_ALL, K)
    dpack = jnp.concatenate([dst, jnp.zeros((pad,), jnp.int32)]).reshape(
        NCH_ALL, K)
    wpack = jnp.concatenate([w, jnp.zeros((pad,), jnp.float32)]).reshape(
        NCH_ALL, K)
    sdwpack = jnp.stack(
        [spack, dpack, lax.bitcast_convert_type(wpack, jnp.int32)], axis=1)

    degp = _deg_kernel(dpack, wpack)
    dinv = _prep(degp.reshape(NC, NPAD // 128, 128))
    dinv_col = dinv.reshape(NPAD)[:N, None]

    h1 = _mm(x, W1, dinv_col)
    part1 = _prop_kernel(h1, sdwpack)
    h2 = _fuse_mm(part1, part1, h1, dinv_col, b1.reshape(1, D), W2)
    part2 = _prop_kernel(h2, sdwpack)
    Wout_pad = jnp.pad(Wout, ((0, 0), (0, 128 - C)))
    bout_pad = jnp.pad(bout, (0, 128 - C)).reshape(1, 128)
    outp = _final(part2, part2, h2, dinv_col, b2.reshape(1, D),
                  Wout_pad, bout_pad)
    return outp[:, :C]


# speedup vs baseline: 9.9484x; 1.0905x over previous
import functools

import jax
import jax.numpy as jnp
from jax import lax
from jax.experimental import pallas as pl
from jax.experimental.pallas import tpu as pltpu
from jax.experimental.pallas import tpu_sc as plsc

N = 10000
E = 320000
D = 128
C = 64
NC, NS, LANES = 2, 16, 16
NW = NC * NS
K = 128
CPW = 80
EPW = K * CPW
EPAD = EPW * NW
NCH_ALL = EPAD // K
NPAD = 10240
NPT = NPAD // NS
MMB = 400

_SC_PARAMS = pltpu.CompilerParams(needs_layout_passes=False)


def _mesh():
    return plsc.VectorSubcoreMesh(core_axis_name="c", subcore_axis_name="s")


def _splat(v):
    return jnp.zeros((LANES,), jnp.int32) + v


@functools.partial(
    pl.kernel,
    out_type=jax.ShapeDtypeStruct((NC, NPAD), jnp.float32),
    mesh=_mesh(),
    compiler_params=_SC_PARAMS,
    scratch_types=[
        pltpu.VMEM((NPAD,), jnp.float32),
        pltpu.VMEM((CPW, K), jnp.int32),
        pltpu.VMEM((CPW, K), jnp.float32),
        pltpu.VMEM((NS, NPT), jnp.float32),
        pltpu.VMEM_SHARED((NS, NPAD), jnp.float32),
    ],
)
def _deg_kernel(dpack_hbm, wpack_hbm, degp, acc, dbig, wbig, rbuf, shared):
    cid = lax.axis_index("c")
    sid = lax.axis_index("s")
    wid = cid * NS + sid

    def zero_body(i, c):
        acc[pl.ds(i * LANES, LANES)] = jnp.zeros((LANES,), jnp.float32)
        return c

    lax.fori_loop(0, NPAD // LANES, zero_body, 0)
    slab = pl.ds(pl.multiple_of(wid * CPW, 8), CPW)
    pltpu.sync_copy(dpack_hbm.at[slab], dbig)
    pltpu.sync_copy(wpack_hbm.at[slab], wbig)

    def chunk_body(c, carry):
        for i in range(K // LANES):
            idx = dbig[c, pl.ds(i * LANES, LANES)]
            val = wbig[c, pl.ds(i * LANES, LANES)]
            plsc.addupdate_scatter(acc, [idx], val)
        return carry

    lax.fori_loop(0, CPW, chunk_body, 0)

    pltpu.sync_copy(acc, shared.at[sid])
    plsc.subcore_barrier()
    col0 = pl.multiple_of(sid * NPT, 8)
    for r in range(NS):
        pltpu.sync_copy(shared.at[r, pl.ds(col0, NPT)], rbuf.at[r])

    def red_body(i, carry):
        s = rbuf[0, pl.ds(i * LANES, LANES)]
        for r in range(1, NS):
            s = s + rbuf[r, pl.ds(i * LANES, LANES)]
        acc[pl.ds(i * LANES, LANES)] = s
        return carry

    lax.fori_loop(0, NPT // LANES, red_body, 0)
    pltpu.sync_copy(acc.at[pl.ds(0, NPT)], degp.at[cid, pl.ds(col0, NPT)])


@functools.partial(
    pl.kernel,
    out_type=jax.ShapeDtypeStruct((NC, NPAD, D), jnp.float32),
    mesh=_mesh(),
    compiler_params=_SC_PARAMS,
    scratch_types=[
        [pltpu.VMEM((K,), jnp.int32) for _ in range(2)],
        [pltpu.VMEM((K,), jnp.int32) for _ in range(2)],
        pltpu.VMEM((CPW, K), jnp.float32),
        [pltpu.VMEM((K, D), jnp.float32) for _ in range(2)],
        pltpu.VMEM((32, D), jnp.float32),
        pltpu.VMEM_SHARED((NPAD, D), jnp.float32),
        [pltpu.SemaphoreType.DMA for _ in range(2)],
    ],
)
def _prop_kernel(h_hbm, spack_hbm, dpack_hbm, wpack_hbm, part,
                 sidx, didx, wbig, rows, zbuf, acc, gsem):
    cid = lax.axis_index("c")
    sid = lax.axis_index("s")
    wid = cid * NS + sid

    def zfill(i, c):
        for j in range(D // LANES):
            zbuf[i, pl.ds(j * LANES, LANES)] = jnp.zeros((LANES,), jnp.float32)
        return c

    lax.fori_loop(0, 32, zfill, 0)
    row0 = pl.multiple_of(sid * NPT, 8)

    def zcopy(k, carry):
        pltpu.sync_copy(zbuf, acc.at[pl.ds(pl.multiple_of(row0 + k * 32, 8),
                                           32)])
        return carry

    lax.fori_loop(0, NPT // 32, zcopy, 0)
    plsc.subcore_barrier()

    slab = pl.ds(pl.multiple_of(wid * CPW, 8), CPW)
    pltpu.sync_copy(wpack_hbm.at[slab], wbig)
    ebase = wid * EPW

    def gather_start(c, b):
        off = pl.multiple_of(ebase + c * K, 8)
        pltpu.sync_copy(spack_hbm.at[pl.ds(off, K)], sidx[b])
        pltpu.sync_copy(dpack_hbm.at[pl.ds(off, K)], didx[b])
        pltpu.async_copy(h_hbm.at[sidx[b]], rows[b], gsem[b])

    def gather_wait(c, b):
        pltpu.make_async_copy(h_hbm.at[sidx[b]], rows[b], gsem[b]).wait()

    def process(c, b):
        gather_wait(c, b)
        c16 = _splat(c)

        def sbody(e, cc):
            s = plsc.load_gather(wbig, [c16, _splat(e)])
            for j in range(D // LANES):
                rows[b][e, pl.ds(j * LANES, LANES)] = (
                    rows[b][e, pl.ds(j * LANES, LANES)] * s)
            return cc

        lax.fori_loop(0, K, sbody, 0)
        pltpu.sync_copy(rows[b], acc.at[didx[b]], add=True)

    gather_start(0, 0)

    def loop_body(i, carry):
        c = i * 2
        gather_start(c + 1, 1)
        process(c, 0)

        @pl.when(c + 2 < CPW)
        def _():
            gather_start(c + 2, 0)

        process(c + 1, 1)
        return carry

    lax.fori_loop(0, CPW // 2, loop_body, 0)
    plsc.subcore_barrier()

    def wb_body(k, carry):
        r = pl.multiple_of(row0 + k * 64, 8)
        pltpu.sync_copy(acc.at[pl.ds(r, 64)], part.at[cid, pl.ds(r, 64)])
        return carry

    lax.fori_loop(0, NPT // 64, wb_body, 0)


def _prep_body(degp_ref, dinv_ref):
    deg = degp_ref[0] + degp_ref[1] + 1.0
    dinv_ref[...] = lax.rsqrt(deg)


_prep = pl.pallas_call(
    _prep_body,
    out_shape=jax.ShapeDtypeStruct((NPAD // 128, 128), jnp.float32),
)


def _mm_body(a_ref, w_ref, dv_ref, o_ref):
    o_ref[...] = dv_ref[...] * jnp.dot(a_ref[...], w_ref[...],
                                       preferred_element_type=jnp.float32,
                                       precision=lax.Precision.HIGHEST)


_mm = pl.pallas_call(
    _mm_body,
    grid=(N // MMB,),
    in_specs=[pl.BlockSpec((MMB, D), lambda i: (i, 0)),
              pl.BlockSpec((D, D), lambda i: (0, 0)),
              pl.BlockSpec((MMB, 1), lambda i: (i, 0))],
    out_specs=pl.BlockSpec((MMB, D), lambda i: (i, 0)),
    out_shape=jax.ShapeDtypeStruct((N, D), jnp.float32),
)


def _fuse_mm_body(p0_ref, p1_ref, h_ref, dv_ref, b_ref, w_ref, o_ref):
    a = dv_ref[...] * (p0_ref[0] + p1_ref[0] + h_ref[...]) + b_ref[...]
    a = jnp.maximum(a, 0.0)
    o_ref[...] = dv_ref[...] * jnp.dot(a, w_ref[...],
                                       preferred_element_type=jnp.float32,
                                       precision=lax.Precision.HIGHEST)


_fuse_mm = pl.pallas_call(
    _fuse_mm_body,
    grid=(N // MMB,),
    in_specs=[pl.BlockSpec((1, MMB, D), lambda i: (0, i, 0)),
              pl.BlockSpec((1, MMB, D), lambda i: (1, i, 0)),
              pl.BlockSpec((MMB, D), lambda i: (i, 0)),
              pl.BlockSpec((MMB, 1), lambda i: (i, 0)),
              pl.BlockSpec((1, D), lambda i: (0, 0)),
              pl.BlockSpec((D, D), lambda i: (0, 0))],
    out_specs=pl.BlockSpec((MMB, D), lambda i: (i, 0)),
    out_shape=jax.ShapeDtypeStruct((N, D), jnp.float32),
)


def _final_body(p0_ref, p1_ref, h_ref, dv_ref, b_ref, w_ref, bo_ref, o_ref):
    a = dv_ref[...] * (p0_ref[0] + p1_ref[0] + h_ref[...]) + b_ref[...]
    a = jnp.maximum(a, 0.0)
    logits = jnp.dot(a, w_ref[...],
                     preferred_element_type=jnp.float32,
                     precision=lax.Precision.HIGHEST) + bo_ref[...]
    col = lax.broadcasted_iota(jnp.int32, (MMB, 128), 1)
    lm = jnp.where(col < C, logits, jnp.float32(-1e30))
    m = jnp.max(lm, axis=1, keepdims=True)
    ex = jnp.where(col < C, jnp.exp(lm - m), 0.0)
    o_ref[...] = ex / jnp.sum(ex, axis=1, keepdims=True)


_final = pl.pallas_call(
    _final_body,
    grid=(N // MMB,),
    in_specs=[pl.BlockSpec((1, MMB, D), lambda i: (0, i, 0)),
              pl.BlockSpec((1, MMB, D), lambda i: (1, i, 0)),
              pl.BlockSpec((MMB, D), lambda i: (i, 0)),
              pl.BlockSpec((MMB, 1), lambda i: (i, 0)),
              pl.BlockSpec((1, D), lambda i: (0, 0)),
              pl.BlockSpec((D, 128), lambda i: (0, 0)),
              pl.BlockSpec((1, 128), lambda i: (0, 0))],
    out_specs=pl.BlockSpec((MMB, 128), lambda i: (i, 0)),
    out_shape=jax.ShapeDtypeStruct((N, 128), jnp.float32),
)


def kernel(x, edge_index, edge_weight, W1, b1, W2, b2, Wout, bout):
    src = edge_index[0].astype(jnp.int32)
    dst = edge_index[1].astype(jnp.int32)
    w = edge_weight.astype(jnp.float32)
    pad = EPAD - E
    spack = jnp.concatenate([src, jnp.zeros((pad,), jnp.int32)]).reshape(
        NCH_ALL, K)
    dpack = jnp.concatenate([dst, jnp.zeros((pad,), jnp.int32)]).reshape(
        NCH_ALL, K)
    wpack = jnp.concatenate([w, jnp.zeros((pad,), jnp.float32)]).reshape(
        NCH_ALL, K)
    degp = _deg_kernel(dpack, wpack)
    dinv = _prep(degp.reshape(NC, NPAD // 128, 128))
    dinv_col = dinv.reshape(NPAD)[:N, None]

    h1 = _mm(x, W1, dinv_col)
    part1 = _prop_kernel(h1, spack.reshape(EPAD), dpack.reshape(EPAD), wpack)
    h2 = _fuse_mm(part1, part1, h1, dinv_col, b1.reshape(1, D), W2)
    part2 = _prop_kernel(h2, spack.reshape(EPAD), dpack.reshape(EPAD), wpack)
    Wout_pad = jnp.pad(Wout, ((0, 0), (0, 128 - C)))
    bout_pad = jnp.pad(bout, (0, 128 - C)).reshape(1, 128)
    outp = _final(part2, part2, h2, dinv_col, b2.reshape(1, D),
                  Wout_pad, bout_pad)
    return outp[:, :C]
